# Initial kernel scaffold; baseline (speedup 1.0000x reference)
import jax
import jax.numpy as jnp
from jax.experimental import pallas as pl


def kernel(adj_indices, adj_values, embeds, qTrans, kTrans, vTrans):
    raise NotImplementedError("write your pallas kernel here")


# speedup vs baseline: 3.4961x; 3.4961x over previous
import functools

import jax
import jax.numpy as jnp
from jax import lax
from jax.experimental import pallas as pl
from jax.experimental.pallas import tpu as pltpu
from jax.experimental.pallas import tpu_sc as plsc

NC = 2
NS = 16
NW = NC * NS
L = 16

CHUNK = 128


def _dense_body(e_ref, q_ref, k_ref, v_ref, res_ref, att_ref):
    e = e_ref[...]
    q = jnp.dot(e, q_ref[...], preferred_element_type=jnp.float32)
    k = jnp.dot(e, k_ref[...], preferred_element_type=jnp.float32)
    v = jnp.dot(e, v_ref[...], preferred_element_type=jnp.float32)
    att = jnp.sum(q * k, axis=1, keepdims=True)
    att = jnp.clip(att, -10.0, 10.0)
    ex = jnp.exp(att)
    an = ex / (ex + 1e-8)
    res_ref[...] = an * v
    att_ref[...] = an


def _dense(embeds, qT, kT, vT):
    n, d = embeds.shape
    rb = 1000
    grid = n // rb
    return pl.pallas_call(
        _dense_body,
        grid=(grid,),
        in_specs=[
            pl.BlockSpec((rb, d), lambda i: (i, 0)),
            pl.BlockSpec((d, d), lambda i: (0, 0)),
            pl.BlockSpec((d, d), lambda i: (0, 0)),
            pl.BlockSpec((d, d), lambda i: (0, 0)),
        ],
        out_specs=[
            pl.BlockSpec((rb, d), lambda i: (i, 0)),
            pl.BlockSpec((rb, 1), lambda i: (i, 0)),
        ],
        out_shape=[
            jax.ShapeDtypeStruct((n, d), jnp.float32),
            jax.ShapeDtypeStruct((n, 1), jnp.float32),
        ],
    )(embeds, qT, kT, vT)


def _spmm_body(n, ept, res_hbm, row_hbm, col_hbm, val_hbm, out_hbm,
               col_v, row_v, val_v, rows_v, zero_v, acc_sh, sem_g):
    c = lax.axis_index("c")
    s = lax.axis_index("s")
    wid = s * NC + c
    d = res_hbm.shape[1]
    ndv = d // L

    zrows = zero_v.shape[0]
    for j in range(zrows):
        for g in range(ndv):
            zero_v[j, pl.ds(g * L, L)] = jnp.zeros((L,), jnp.float32)
    rpt = (n // NS) // 8 * 8
    tail = n - NS * rpt
    nz = rpt // zrows

    def zero_step(i, _):
        pltpu.sync_copy(zero_v, acc_sh.at[pl.ds(s * rpt + i * zrows, zrows)])
        return _

    lax.fori_loop(0, nz, zero_step, None)

    @pl.when(jnp.logical_and(s == NS - 1, tail > 0))
    def _():
        pltpu.sync_copy(zero_v.at[pl.ds(0, tail)],
                        acc_sh.at[pl.ds(NS * rpt, tail)])

    plsc.subcore_barrier()

    base_e = wid * ept
    nchunks = ept // CHUNK

    def chunk_step(j, _):
        off = base_e + j * CHUNK
        pltpu.sync_copy(col_hbm.at[pl.ds(off, CHUNK)], col_v)
        pltpu.sync_copy(row_hbm.at[pl.ds(off, CHUNK)], row_v)
        pltpu.sync_copy(val_hbm.at[pl.ds(off, CHUNK)], val_v)
        pltpu.async_copy(res_hbm.at[col_v], rows_v, sem_g).wait()

        def mul_step(e, _):
            vs = plsc.load_gather(
                val_v, [jnp.broadcast_to(e, (L,)).astype(jnp.int32)])
            for g in range(ndv):
                rows_v[e, pl.ds(g * L, L)] = rows_v[e, pl.ds(g * L, L)] * vs
            return _

        lax.fori_loop(0, CHUNK, mul_step, None)
        pltpu.sync_copy(rows_v, acc_sh.at[row_v], add=True)
        return _

    lax.fori_loop(0, nchunks, chunk_step, None)
    plsc.subcore_barrier()

    pltpu.sync_copy(acc_sh.at[pl.ds(s * rpt, rpt)],
                    out_hbm.at[c, pl.ds(s * rpt, rpt)])

    @pl.when(jnp.logical_and(s == NS - 1, tail > 0))
    def _():
        pltpu.sync_copy(acc_sh.at[pl.ds(NS * rpt, tail)],
                        out_hbm.at[c, pl.ds(NS * rpt, tail)])


def _spmm(res, rows, cols, vals):
    n, d = res.shape
    epad = rows.shape[0]
    ept = epad // NW
    zrows = 24
    mesh = plsc.VectorSubcoreMesh(core_axis_name="c", subcore_axis_name="s")
    kern = pl.kernel(
        functools.partial(_spmm_body, n, ept),
        out_type=jax.ShapeDtypeStruct((NC, n, d), jnp.float32),
        mesh=mesh,
        scratch_types=[
            pltpu.VMEM((CHUNK,), jnp.int32),
            pltpu.VMEM((CHUNK,), jnp.int32),
            pltpu.VMEM((CHUNK,), jnp.float32),
            pltpu.VMEM((CHUNK, d), jnp.float32),
            pltpu.VMEM((zrows, d), jnp.float32),
            pltpu.VMEM_SHARED((n, d), jnp.float32),
            pltpu.SemaphoreType.DMA,
        ],
        compiler_params=pltpu.CompilerParams(needs_layout_passes=False),
    )
    return kern(res, rows, cols, vals)


def _combine_body(p_ref, o_ref):
    o_ref[...] = p_ref[0] + p_ref[1]


def _combine(partials):
    _, n, d = partials.shape
    rb = 1000
    return pl.pallas_call(
        _combine_body,
        grid=(n // rb,),
        in_specs=[pl.BlockSpec((2, rb, d), lambda i: (0, i, 0))],
        out_specs=pl.BlockSpec((rb, d), lambda i: (i, 0)),
        out_shape=jax.ShapeDtypeStruct((n, d), jnp.float32),
    )(partials)


def kernel(adj_indices, adj_values, embeds, qTrans, kTrans, vTrans):
    n, d = embeds.shape
    e = adj_values.shape[0]

    res, att_norm = _dense(embeds, qTrans, kTrans, vTrans)

    step = NW * CHUNK
    epad = ((e + step - 1) // step) * step
    rows = jnp.zeros((epad,), jnp.int32).at[:e].set(
        adj_indices[0].astype(jnp.int32))
    cols = jnp.zeros((epad,), jnp.int32).at[:e].set(
        adj_indices[1].astype(jnp.int32))
    vals = jnp.zeros((epad,), jnp.float32).at[:e].set(adj_values)

    partials = _spmm(res, rows, cols, vals)
    out = _combine(partials)
    return (out, att_norm)
